# Initial kernel scaffold; baseline (speedup 1.0000x reference)
import jax
import jax.numpy as jnp
from jax.experimental import pallas as pl


def kernel(hidden_states, gate_w, w1, w3, w2):
    raise NotImplementedError("write your pallas kernel here")


# speedup vs baseline: 1.3780x; 1.3780x over previous
import functools

import jax
import jax.numpy as jnp
from jax import lax
from jax.experimental import pallas as pl
from jax.experimental.pallas import tpu as pltpu
from jax.experimental.pallas import tpu_sc as plsc

NUM_EXPERTS = 8
TOPK = 2
HID = 1024
FFN = 2048
TOKENS = 2048
JITTER = 0.01

BLK = 256
PADDED = TOKENS * TOPK + NUM_EXPERTS * BLK
NBLOCKS = PADDED // BLK


def _routing_body(x_ref, gw_ref, logits_ref, mw_ref, ids_ref):
    x = x_ref[...]
    gw = gw_ref[...]
    scores = lax.dot_general(x, gw, (((1,), (1,)), ((), ())),
                             preferred_element_type=jnp.float32)
    logits_ref[...] = scores

    t, e = scores.shape
    col = lax.broadcasted_iota(jnp.int32, (t, e), 1)
    neg = -jnp.inf

    thr = jnp.max(scores, axis=-1, keepdims=True)
    max_ind = jnp.min(jnp.where(scores == thr, col, e), axis=-1, keepdims=True)
    factor = jnp.maximum(jnp.abs(scores), thr)
    mask = (thr - scores) / factor > 2 * JITTER
    mg = jnp.where(mask, neg, scores)
    mg = jax.nn.softmax(mg, axis=-1)
    sel1 = col == max_ind
    mult1 = jnp.sum(jnp.where(sel1, mg, 0.0), axis=-1, keepdims=True)

    masked_scores = jnp.where(sel1, neg, scores)
    thr2 = jnp.max(masked_scores, axis=-1, keepdims=True)
    max_ind2 = jnp.min(jnp.where(masked_scores == thr2, col, e),
                       axis=-1, keepdims=True)
    factor2 = jnp.maximum(jnp.abs(scores), thr2)
    mask2 = (thr2 - scores) / factor2 > 2 * JITTER
    mg2 = jnp.where(mask2, neg, masked_scores)
    mg2 = jax.nn.softmax(mg2, axis=-1)
    sel2 = col == max_ind2
    mult2 = jnp.sum(jnp.where(sel2, mg2, 0.0), axis=-1, keepdims=True)

    mw_ref[...] = jnp.where(col == 0, mult1, jnp.where(col == 1, mult2, 0.0))
    ids_ref[...] = jnp.where(col == 0, max_ind,
                             jnp.where(col == 1, max_ind2, 0))


def _routing_call(x, gate_w):
    return pl.pallas_call(
        _routing_body,
        out_shape=[
            jax.ShapeDtypeStruct((TOKENS, NUM_EXPERTS), jnp.float32),
            jax.ShapeDtypeStruct((TOKENS, NUM_EXPERTS), jnp.float32),
            jax.ShapeDtypeStruct((TOKENS, NUM_EXPERTS), jnp.int32),
        ],
    )(x, gate_w)


def _dispatch_plan(ids, weights):
    e_flat = ids.T.reshape(-1)
    w_flat = weights.T.reshape(-1)
    tok = jnp.tile(jnp.arange(TOKENS, dtype=jnp.int32), TOPK)

    counts = jnp.bincount(e_flat, length=NUM_EXPERTS)
    padded_counts = ((counts + BLK - 1) // BLK) * BLK
    ends = jnp.cumsum(padded_counts)
    starts = ends - padded_counts

    onehot = (e_flat[:, None] == jnp.arange(NUM_EXPERTS)[None, :])
    ranks = jnp.cumsum(onehot.astype(jnp.int32), axis=0) - 1
    rank = jnp.take_along_axis(ranks, e_flat[:, None], axis=1)[:, 0]
    pos = (starts[e_flat] + rank).astype(jnp.int32)

    rows = jnp.zeros((PADDED,), jnp.int32).at[pos].set(tok)
    wrow = jnp.zeros((PADDED,), jnp.float32).at[pos].set(w_flat)
    block_expert = jnp.searchsorted(
        ends, jnp.arange(NBLOCKS, dtype=jnp.int32) * BLK, side="right")
    block_expert = jnp.minimum(block_expert, NUM_EXPERTS - 1).astype(jnp.int32)
    p0 = pos[:TOKENS]
    p1 = pos[TOKENS:]
    return rows, wrow, block_expert, p0, p1


def _sc_gather_call(x, rows):
    info = plsc.get_sparse_core_info()
    nc, ns = info.num_cores, info.num_subcores
    nw = nc * ns
    rpw = PADDED // nw
    ch = min(rpw, 96)
    nch = rpw // ch
    mesh = plsc.VectorSubcoreMesh(core_axis_name="c", subcore_axis_name="s")

    @functools.partial(
        pl.kernel,
        out_type=jax.ShapeDtypeStruct((PADDED, HID), jnp.float32),
        mesh=mesh,
        scratch_types=[
            pltpu.VMEM((ch,), jnp.int32),
            pltpu.VMEM((ch, HID), jnp.float32),
            pltpu.SemaphoreType.DMA,
        ],
    )
    def gather_kernel(x_hbm, rows_hbm, xs_hbm, idx_v, buf_v, sem):
        wid = lax.axis_index("s") * nc + lax.axis_index("c")
        base = wid * rpw

        def chunk(c, carry):
            off = base + c * ch
            pltpu.sync_copy(rows_hbm.at[pl.ds(off, ch)], idx_v)
            pltpu.async_copy(x_hbm.at[idx_v], buf_v, sem).wait()
            pltpu.sync_copy(buf_v, xs_hbm.at[pl.ds(off, ch)])
            return carry

        lax.fori_loop(0, nch, chunk, 0)

    return gather_kernel(x, rows)


def _expert_body(be_ref, xs_ref, w1_ref, w3_ref, w2_ref, wr_ref, ys_ref):
    xs = xs_ref[...]
    w1 = w1_ref[0]
    w3 = w3_ref[0]
    w2 = w2_ref[0]
    h1 = lax.dot_general(xs, w1, (((1,), (1,)), ((), ())),
                         preferred_element_type=jnp.float32)
    h3 = lax.dot_general(xs, w3, (((1,), (1,)), ((), ())),
                         preferred_element_type=jnp.float32)
    h = (h1 * jax.nn.sigmoid(h1)) * h3
    h = h * wr_ref[...]
    ys_ref[...] = lax.dot_general(h, w2, (((1,), (1,)), ((), ())),
                                  preferred_element_type=jnp.float32)


def _expert_call(block_expert, xs, w1, w3, w2, wrow):
    grid_spec = pltpu.PrefetchScalarGridSpec(
        num_scalar_prefetch=1,
        grid=(NBLOCKS,),
        in_specs=[
            pl.BlockSpec((BLK, HID), lambda b, be: (b, 0)),
            pl.BlockSpec((1, FFN, HID), lambda b, be: (be[b], 0, 0)),
            pl.BlockSpec((1, FFN, HID), lambda b, be: (be[b], 0, 0)),
            pl.BlockSpec((1, HID, FFN), lambda b, be: (be[b], 0, 0)),
            pl.BlockSpec((BLK, 1), lambda b, be: (b, 0)),
        ],
        out_specs=pl.BlockSpec((BLK, HID), lambda b, be: (b, 0)),
    )
    return pl.pallas_call(
        _expert_body,
        grid_spec=grid_spec,
        out_shape=jax.ShapeDtypeStruct((PADDED, HID), jnp.float32),
    )(block_expert, xs, w1, w3, w2, wrow[:, None])


def _sc_combine_call(ysw, p0, p1):
    info = plsc.get_sparse_core_info()
    nc, ns = info.num_cores, info.num_subcores
    nw = nc * ns
    tpw = TOKENS // nw
    ch = min(tpw, 32)
    nch = tpw // ch
    mesh = plsc.VectorSubcoreMesh(core_axis_name="c", subcore_axis_name="s")

    @functools.partial(
        pl.kernel,
        out_type=jax.ShapeDtypeStruct((TOKENS, HID), jnp.float32),
        mesh=mesh,
        scratch_types=[
            pltpu.VMEM((ch,), jnp.int32),
            pltpu.VMEM((ch,), jnp.int32),
            pltpu.VMEM((ch, HID), jnp.float32),
            pltpu.VMEM((ch, HID), jnp.float32),
            pltpu.SemaphoreType.DMA,
            pltpu.SemaphoreType.DMA,
        ],
    )
    def combine_kernel(ys_hbm, p0_hbm, p1_hbm, out_hbm,
                       i0_v, i1_v, buf0, buf1, sem0, sem1):
        wid = lax.axis_index("s") * nc + lax.axis_index("c")
        base = wid * tpw

        def chunk(c, carry):
            off = base + c * ch
            pltpu.sync_copy(p0_hbm.at[pl.ds(off, ch)], i0_v)
            pltpu.sync_copy(p1_hbm.at[pl.ds(off, ch)], i1_v)
            cp0 = pltpu.async_copy(ys_hbm.at[i0_v], buf0, sem0)
            cp1 = pltpu.async_copy(ys_hbm.at[i1_v], buf1, sem1)
            cp0.wait()
            cp1.wait()

            def addrow(r, carry2):
                row0 = buf0.at[r]
                row1 = buf1.at[r]
                for j in range(HID // 16):
                    sl = pl.ds(j * 16, 16)
                    row0[sl] = row0[sl] + row1[sl]
                return carry2

            lax.fori_loop(0, ch, addrow, 0)
            pltpu.sync_copy(buf0, out_hbm.at[pl.ds(off, ch)])
            return carry

        lax.fori_loop(0, nch, chunk, 0)

    return combine_kernel(ysw, p0, p1)


def kernel(hidden_states, gate_w, w1, w3, w2):
    b, s, hd = hidden_states.shape
    x = hidden_states.reshape(-1, hd)

    logits, mw, ids8 = _routing_call(x, gate_w)
    ids = ids8[:, :TOPK]
    wts = mw[:, :TOPK]
    rows, wrow, block_expert, p0, p1 = _dispatch_plan(ids, wts)

    xs = _sc_gather_call(x, rows)
    ysw = _expert_call(block_expert, xs, w1, w3, w2, wrow)
    out = _sc_combine_call(ysw, p0, p1)

    return out.reshape(b, s, hd), logits


# speedup vs baseline: 1.4288x; 1.0369x over previous
import functools

import jax
import jax.numpy as jnp
from jax import lax
from jax.experimental import pallas as pl
from jax.experimental.pallas import tpu as pltpu
from jax.experimental.pallas import tpu_sc as plsc

NUM_EXPERTS = 8
TOPK = 2
HID = 1024
FFN = 2048
TOKENS = 2048
JITTER = 0.01

BLK = 256
PADDED = TOKENS * TOPK + NUM_EXPERTS * BLK
NBLOCKS = PADDED // BLK


def _routing_body(x_ref, gw_ref, logits_ref, mw_ref, ids_ref):
    x = x_ref[...]
    gw = gw_ref[...]
    scores = lax.dot_general(x, gw, (((1,), (1,)), ((), ())),
                             preferred_element_type=jnp.float32)
    logits_ref[...] = scores

    t, e = scores.shape
    col = lax.broadcasted_iota(jnp.int32, (t, e), 1)
    neg = -jnp.inf

    thr = jnp.max(scores, axis=-1, keepdims=True)
    max_ind = jnp.min(jnp.where(scores == thr, col, e), axis=-1, keepdims=True)
    factor = jnp.maximum(jnp.abs(scores), thr)
    mask = (thr - scores) / factor > 2 * JITTER
    mg = jnp.where(mask, neg, scores)
    mg = jax.nn.softmax(mg, axis=-1)
    sel1 = col == max_ind
    mult1 = jnp.sum(jnp.where(sel1, mg, 0.0), axis=-1, keepdims=True)

    masked_scores = jnp.where(sel1, neg, scores)
    thr2 = jnp.max(masked_scores, axis=-1, keepdims=True)
    max_ind2 = jnp.min(jnp.where(masked_scores == thr2, col, e),
                       axis=-1, keepdims=True)
    factor2 = jnp.maximum(jnp.abs(scores), thr2)
    mask2 = (thr2 - scores) / factor2 > 2 * JITTER
    mg2 = jnp.where(mask2, neg, masked_scores)
    mg2 = jax.nn.softmax(mg2, axis=-1)
    sel2 = col == max_ind2
    mult2 = jnp.sum(jnp.where(sel2, mg2, 0.0), axis=-1, keepdims=True)

    mw_ref[...] = jnp.where(col == 0, mult1, jnp.where(col == 1, mult2, 0.0))
    ids_ref[...] = jnp.where(col == 0, max_ind,
                             jnp.where(col == 1, max_ind2, 0))


def _routing_call(x, gate_w):
    return pl.pallas_call(
        _routing_body,
        out_shape=[
            jax.ShapeDtypeStruct((TOKENS, NUM_EXPERTS), jnp.float32),
            jax.ShapeDtypeStruct((TOKENS, NUM_EXPERTS), jnp.float32),
            jax.ShapeDtypeStruct((TOKENS, NUM_EXPERTS), jnp.int32),
        ],
    )(x, gate_w)


def _dispatch_plan(ids, weights):
    e_flat = ids.T.reshape(-1)
    w_flat = weights.T.reshape(-1)
    tok = jnp.tile(jnp.arange(TOKENS, dtype=jnp.int32), TOPK)

    counts = jnp.bincount(e_flat, length=NUM_EXPERTS)
    padded_counts = ((counts + BLK - 1) // BLK) * BLK
    ends = jnp.cumsum(padded_counts)
    starts = ends - padded_counts

    onehot = (e_flat[:, None] == jnp.arange(NUM_EXPERTS)[None, :])
    ranks = jnp.cumsum(onehot.astype(jnp.int32), axis=0) - 1
    rank = jnp.take_along_axis(ranks, e_flat[:, None], axis=1)[:, 0]
    pos = (starts[e_flat] + rank).astype(jnp.int32)

    rows = jnp.zeros((PADDED,), jnp.int32).at[pos].set(tok)
    wrow = jnp.zeros((PADDED,), jnp.float32).at[pos].set(w_flat)
    block_starts = jnp.arange(NBLOCKS, dtype=jnp.int32) * BLK
    block_expert = jnp.searchsorted(ends, block_starts, side="right")
    block_expert = jnp.minimum(block_expert, NUM_EXPERTS - 1).astype(jnp.int32)
    block_active = (block_starts < ends[-1]).astype(jnp.int32)
    p0 = pos[:TOKENS]
    p1 = pos[TOKENS:]
    return rows, wrow, block_expert, block_active, p0, p1


def _sc_gather_call(x, rows):
    info = plsc.get_sparse_core_info()
    nc, ns = info.num_cores, info.num_subcores
    nw = nc * ns
    rpw = PADDED // nw
    ch = 48
    nch = rpw // ch
    mesh = plsc.VectorSubcoreMesh(core_axis_name="c", subcore_axis_name="s")

    @functools.partial(
        pl.kernel,
        out_type=jax.ShapeDtypeStruct((PADDED, HID), jnp.float32),
        mesh=mesh,
        scratch_types=[
            pltpu.VMEM((rpw,), jnp.int32),
            pltpu.VMEM((ch, HID), jnp.float32),
            pltpu.VMEM((ch, HID), jnp.float32),
            pltpu.SemaphoreType.DMA,
            pltpu.SemaphoreType.DMA,
            pltpu.SemaphoreType.DMA,
            pltpu.SemaphoreType.DMA,
        ],
    )
    def gather_kernel(x_hbm, rows_hbm, xs_hbm, idx_v, buf0, buf1,
                      g0, g1, s0, s1):
        wid = lax.axis_index("s") * nc + lax.axis_index("c")
        base = wid * rpw
        pltpu.sync_copy(rows_hbm.at[pl.ds(base, rpw)], idx_v)
        bufs = (buf0, buf1)
        gsems = (g0, g1)
        wsems = (s0, s1)
        gath = [None] * nch
        wrt = [None] * nch
        gath[0] = pltpu.async_copy(
            x_hbm.at[idx_v.at[pl.ds(0, ch)]], bufs[0], gsems[0])
        for c in range(nch):
            i = c % 2
            gath[c].wait()
            if c + 1 < nch:
                j = (c + 1) % 2
                if c >= 1:
                    wrt[c - 1].wait()
                gath[c + 1] = pltpu.async_copy(
                    x_hbm.at[idx_v.at[pl.ds((c + 1) * ch, ch)]],
                    bufs[j], gsems[j])
            wrt[c] = pltpu.async_copy(
                bufs[i], xs_hbm.at[pl.ds(base + c * ch, ch)], wsems[i])
        if nch >= 2:
            wrt[nch - 2].wait()
        wrt[nch - 1].wait()

    return gather_kernel(x, rows)


def _expert_body(be_ref, act_ref, xs_ref, w1_ref, w3_ref, w2_ref, wr_ref,
                 ys_ref):
    b = pl.program_id(0)

    @pl.when(act_ref[b] == 1)
    def _():
        xs = xs_ref[...]
        w1 = w1_ref[0]
        w3 = w3_ref[0]
        w2 = w2_ref[0]
        h1 = lax.dot_general(xs, w1, (((1,), (1,)), ((), ())),
                             preferred_element_type=jnp.float32)
        h3 = lax.dot_general(xs, w3, (((1,), (1,)), ((), ())),
                             preferred_element_type=jnp.float32)
        h = (h1 * jax.nn.sigmoid(h1)) * h3
        h = h * wr_ref[...]
        ys_ref[...] = lax.dot_general(h, w2, (((1,), (1,)), ((), ())),
                                      preferred_element_type=jnp.float32)


def _expert_call(block_expert, block_active, xs, w1, w3, w2, wrow):
    grid_spec = pltpu.PrefetchScalarGridSpec(
        num_scalar_prefetch=2,
        grid=(NBLOCKS,),
        in_specs=[
            pl.BlockSpec((BLK, HID), lambda b, be, act: (b, 0)),
            pl.BlockSpec((1, FFN, HID), lambda b, be, act: (be[b], 0, 0)),
            pl.BlockSpec((1, FFN, HID), lambda b, be, act: (be[b], 0, 0)),
            pl.BlockSpec((1, HID, FFN), lambda b, be, act: (be[b], 0, 0)),
            pl.BlockSpec((BLK, 1), lambda b, be, act: (b, 0)),
        ],
        out_specs=pl.BlockSpec((BLK, HID), lambda b, be, act: (b, 0)),
    )
    return pl.pallas_call(
        _expert_body,
        grid_spec=grid_spec,
        out_shape=jax.ShapeDtypeStruct((PADDED, HID), jnp.float32),
    )(block_expert, block_active, xs, w1, w3, w2, wrow[:, None])


def _sc_combine_call(ysw, p0, p1):
    info = plsc.get_sparse_core_info()
    nc, ns = info.num_cores, info.num_subcores
    nw = nc * ns
    tpw = TOKENS // nw
    ch = min(tpw, 32)
    nch = tpw // ch
    mesh = plsc.VectorSubcoreMesh(core_axis_name="c", subcore_axis_name="s")

    @functools.partial(
        pl.kernel,
        out_type=jax.ShapeDtypeStruct((TOKENS, HID), jnp.float32),
        mesh=mesh,
        scratch_types=[
            pltpu.VMEM((ch,), jnp.int32),
            pltpu.VMEM((ch,), jnp.int32),
            pltpu.VMEM((ch, HID), jnp.float32),
            pltpu.VMEM((ch, HID), jnp.float32),
            pltpu.SemaphoreType.DMA,
            pltpu.SemaphoreType.DMA,
        ],
    )
    def combine_kernel(ys_hbm, p0_hbm, p1_hbm, out_hbm,
                       i0_v, i1_v, buf0, buf1, sem0, sem1):
        wid = lax.axis_index("s") * nc + lax.axis_index("c")
        base = wid * tpw

        def chunk(c, carry):
            off = base + c * ch
            pltpu.sync_copy(p0_hbm.at[pl.ds(off, ch)], i0_v)
            pltpu.sync_copy(p1_hbm.at[pl.ds(off, ch)], i1_v)
            cp0 = pltpu.async_copy(ys_hbm.at[i0_v], buf0, sem0)
            cp1 = pltpu.async_copy(ys_hbm.at[i1_v], buf1, sem1)
            cp0.wait()
            cp1.wait()

            def addrow(r, carry2):
                row0 = buf0.at[r]
                row1 = buf1.at[r]
                for j in range(HID // 16):
                    sl = pl.ds(j * 16, 16)
                    row0[sl] = row0[sl] + row1[sl]
                return carry2

            lax.fori_loop(0, ch, addrow, 0)
            pltpu.sync_copy(buf0, out_hbm.at[pl.ds(off, ch)])
            return carry

        lax.fori_loop(0, nch, chunk, 0)

    return combine_kernel(ysw, p0, p1)


def kernel(hidden_states, gate_w, w1, w3, w2):
    b, s, hd = hidden_states.shape
    x = hidden_states.reshape(-1, hd)

    logits, mw, ids8 = _routing_call(x, gate_w)
    ids = ids8[:, :TOPK]
    wts = mw[:, :TOPK]
    rows, wrow, block_expert, block_active, p0, p1 = _dispatch_plan(ids, wts)

    xs = _sc_gather_call(x, rows)
    ysw = _expert_call(block_expert, block_active, xs, w1, w3, w2, wrow)
    out = _sc_combine_call(ysw, p0, p1)

    return out.reshape(b, s, hd), logits
